# baseline (speedup 1.0000x reference)
import functools

import numpy as np

import jax
import jax.numpy as jnp
from jax import lax
from jax.experimental import pallas as pl
from jax.experimental.pallas import tpu as pltpu
from jax.experimental.pallas import tpu_sc as plsc

N = 10000
E = 320000
D = 128
NPAD = 10240
NW = 32
CPW = 80
CHUNK = 128
EPAD = NW * CPW * CHUNK
ROWS_PER_TILE = NPAD // 16
GRP = 8
BETA = 0.25
BLK = 1280
BLK1 = 1000

_mesh = plsc.VectorSubcoreMesh(core_axis_name="c", subcore_axis_name="s")


def _tc1_body(x_ref, wfc_ref, wl_ref, wr_ref, ft_ref, sl_ref, sr_ref):
    xb = x_ref[...]
    ft = lax.dot_general(xb, wfc_ref[...], (((1,), (1,)), ((), ())),
                         preferred_element_type=jnp.float32)
    ft_ref[...] = ft
    a1 = lax.dot_general(ft, wl_ref[...], (((1,), (1,)), ((), ())),
                         preferred_element_type=jnp.float32)
    a2 = lax.dot_general(ft, wr_ref[...], (((1,), (1,)), ((), ())),
                         preferred_element_type=jnp.float32)
    sl_ref[...] = jnp.sum(a1, axis=1).reshape(1, BLK // 128, 128)
    sr_ref[...] = jnp.sum(a2, axis=1).reshape(1, BLK // 128, 128)


def _tc_prepare(xp, w_fc, w_l, w_r):
    grid = NPAD // BLK
    return pl.pallas_call(
        _tc1_body,
        grid=(grid,),
        in_specs=[
            pl.BlockSpec((BLK, D), lambda i: (i, 0)),
            pl.BlockSpec((D, D), lambda i: (0, 0)),
            pl.BlockSpec((D, D), lambda i: (0, 0)),
            pl.BlockSpec((D, D), lambda i: (0, 0)),
        ],
        out_specs=[
            pl.BlockSpec((BLK, D), lambda i: (i, 0)),
            pl.BlockSpec((1, BLK // 128, 128), lambda i: (i, 0, 0)),
            pl.BlockSpec((1, BLK // 128, 128), lambda i: (i, 0, 0)),
        ],
        out_shape=[
            jax.ShapeDtypeStruct((NPAD, D), jnp.float32),
            jax.ShapeDtypeStruct((NPAD // BLK, BLK // 128, 128), jnp.float32),
            jax.ShapeDtypeStruct((NPAD // BLK, BLK // 128, 128), jnp.float32),
        ],
    )(xp, w_fc, w_l, w_r)


@functools.partial(
    pl.kernel,
    out_type=[
        jax.ShapeDtypeStruct((NW * CPW, CHUNK), jnp.float32),
        jax.ShapeDtypeStruct((2, NPAD), jnp.float32),
    ],
    mesh=_mesh,
    compiler_params=pltpu.CompilerParams(needs_layout_passes=False),
    scratch_types=[
        pltpu.VMEM((NPAD,), jnp.float32),
        pltpu.VMEM((NPAD,), jnp.float32),
        pltpu.VMEM((CPW, CHUNK), jnp.int32),
        pltpu.VMEM((CPW, CHUNK), jnp.int32),
        pltpu.VMEM((CPW, CHUNK), jnp.float32),
        pltpu.VMEM((CPW, CHUNK), jnp.float32),
        pltpu.VMEM((ROWS_PER_TILE,), jnp.float32),
        pltpu.VMEM_SHARED((NPAD,), jnp.float32),
        pltpu.SemaphoreType.DMA,
    ],
)
def _sc_scores(sl_hbm, sr_hbm, src_hbm, dst_hbm, score_hbm, sp_hbm,
               sl_v, sr_v, dsti, srci, score_v, expb_v, zbuf, s_sh, sem):
    c = lax.axis_index("c")
    s = lax.axis_index("s")
    wid = c * 16 + s
    pltpu.sync_copy(sl_hbm, sl_v)
    pltpu.sync_copy(sr_hbm, sr_v)
    pltpu.sync_copy(dst_hbm.at[pl.ds(wid * CPW, CPW)], dsti)
    pltpu.sync_copy(src_hbm.at[pl.ds(wid * CPW, CPW)], srci)

    def zero(i, _):
        zbuf[pl.ds(i * 16, 16)] = jnp.zeros((16,), jnp.float32)
        return ()
    lax.fori_loop(0, ROWS_PER_TILE // 16, zero, ())
    pltpu.sync_copy(zbuf, s_sh.at[pl.ds(s * ROWS_PER_TILE, ROWS_PER_TILE)])
    plsc.subcore_barrier()

    def chunk(j, _):
        for k in range(8):
            di = dsti[j, pl.ds(k * 16, 16)]
            si = srci[j, pl.ds(k * 16, 16)]
            sc = plsc.load_gather(sl_v, (di,)) + plsc.load_gather(sr_v, (si,))
            sc = jnp.maximum(sc, 0.01 * sc)
            score_v[j, pl.ds(k * 16, 16)] = sc
            expb_v[j, pl.ds(k * 16, 16)] = jnp.exp(BETA * sc)
        pltpu.async_copy(expb_v.at[j], s_sh.at[dsti.at[j]], sem, add=True)
        return ()
    lax.fori_loop(0, CPW, chunk, ())

    def drain(j, _):
        pltpu.make_async_copy(expb_v.at[j], s_sh.at[dsti.at[j]], sem).wait()
        return ()
    lax.fori_loop(0, CPW, drain, ())

    pltpu.sync_copy(score_v, score_hbm.at[pl.ds(wid * CPW, CPW)])
    plsc.subcore_barrier()
    pltpu.sync_copy(s_sh.at[pl.ds(s * ROWS_PER_TILE, ROWS_PER_TILE)],
                    sp_hbm.at[c, pl.ds(s * ROWS_PER_TILE, ROWS_PER_TILE)])


def _tc2_body(sp_ref, c_ref):
    c_ref[...] = jnp.log(sp_ref[0] + sp_ref[1]) * (1.0 / BETA)


def _tc_offset(sp):
    return pl.pallas_call(
        _tc2_body,
        out_shape=jax.ShapeDtypeStruct((NPAD // 128, 128), jnp.float32),
    )(sp)


@functools.partial(
    pl.kernel,
    out_type=[
        jax.ShapeDtypeStruct((2, NPAD, D), jnp.float32),
        jax.ShapeDtypeStruct((2, NPAD), jnp.float32),
    ],
    mesh=_mesh,
    compiler_params=pltpu.CompilerParams(needs_layout_passes=False),
    scratch_types=[
        pltpu.VMEM((NPAD,), jnp.float32),
        pltpu.VMEM((GRP, CHUNK), jnp.int32),
        pltpu.VMEM((GRP, CHUNK), jnp.int32),
        pltpu.VMEM((GRP, CHUNK), jnp.float32),
        pltpu.VMEM((GRP, CHUNK), jnp.float32),
        pltpu.VMEM((ROWS_PER_TILE,), jnp.float32),
        pltpu.VMEM((CHUNK, D), jnp.float32),
        pltpu.VMEM((CHUNK, D), jnp.float32),
        pltpu.VMEM_SHARED((NPAD, D), jnp.float32),
        pltpu.VMEM_SHARED((NPAD,), jnp.float32),
        pltpu.SemaphoreType.DMA,
        pltpu.SemaphoreType.DMA,
        pltpu.SemaphoreType.DMA,
        pltpu.SemaphoreType.DMA,
        pltpu.SemaphoreType.DMA,
    ],
)
def _sc_aggregate(c_hbm, src_hbm, dst_hbm, score_hbm, ft_hbm, ap_hbm, dp_hbm,
                  c_v, dsti, srci, score_v, ex_v, zbuf, rows0, rows1,
                  a_sh, d_sh, gsem0, gsem1, ssem0, ssem1, dsem):
    c = lax.axis_index("c")
    s = lax.axis_index("s")
    wid = c * 16 + s
    pltpu.sync_copy(c_hbm, c_v)

    def zero_row(i, _):
        for q in range(D // 16):
            rows0[i, pl.ds(q * 16, 16)] = jnp.zeros((16,), jnp.float32)
        return ()
    lax.fori_loop(0, CHUNK, zero_row, ())

    def zero1(i, _):
        zbuf[pl.ds(i * 16, 16)] = jnp.zeros((16,), jnp.float32)
        return ()
    lax.fori_loop(0, ROWS_PER_TILE // 16, zero1, ())
    base = s * ROWS_PER_TILE
    for i in range(ROWS_PER_TILE // CHUNK):
        pltpu.sync_copy(rows0, a_sh.at[pl.ds(base + i * CHUNK, CHUNK)])
    pltpu.sync_copy(zbuf, d_sh.at[pl.ds(base, ROWS_PER_TILE)])
    plsc.subcore_barrier()

    def gather_start(j, buf, sem):
        pltpu.async_copy(ft_hbm.at[srci.at[j]], buf, sem)

    def gather_wait(j, buf, sem):
        pltpu.make_async_copy(ft_hbm.at[srci.at[j]], buf, sem).wait()

    def weights(j):
        for k in range(8):
            di = dsti[j, pl.ds(k * 16, 16)]
            cg = plsc.load_gather(c_v, (di,))
            sc = score_v[j, pl.ds(k * 16, 16)]
            ex_v[j, pl.ds(k * 16, 16)] = jnp.exp(sc - cg)
        pltpu.async_copy(ex_v.at[j], d_sh.at[dsti.at[j]], dsem, add=True)

    def scale(j, buf):
        def scale_grp(k, _):
            e16 = ex_v[j, pl.ds(k * 16, 16)]
            for lane in range(16):
                es = e16[lane]
                r = k * 16 + lane
                for q in range(D // 16):
                    buf[r, pl.ds(q * 16, 16)] = buf[r, pl.ds(q * 16, 16)] * es
            return ()
        lax.fori_loop(0, CHUNK // 16, scale_grp, ())

    def scatter_start(j, buf, sem):
        pltpu.async_copy(buf, a_sh.at[dsti.at[j]], sem, add=True)

    def scatter_wait(j, buf, sem):
        pltpu.make_async_copy(buf, a_sh.at[dsti.at[j]], sem).wait()

    def group(g, _):
        gb = wid * CPW + g * GRP
        pltpu.sync_copy(dst_hbm.at[pl.ds(gb, GRP)], dsti)
        pltpu.sync_copy(src_hbm.at[pl.ds(gb, GRP)], srci)
        pltpu.sync_copy(score_hbm.at[pl.ds(gb, GRP)], score_v)
        gather_start(0, rows0, gsem0)

        def pair(i, _):
            j0 = 2 * i
            j1 = j0 + 1
            gather_start(j1, rows1, gsem1)
            weights(j0)
            gather_wait(j0, rows0, gsem0)
            scale(j0, rows0)
            scatter_start(j0, rows0, ssem0)
            scatter_wait(j0, rows0, ssem0)
            gather_start(j0 + 2, rows0, gsem0)
            weights(j1)
            gather_wait(j1, rows1, gsem1)
            scale(j1, rows1)
            scatter_start(j1, rows1, ssem1)
            scatter_wait(j1, rows1, ssem1)
            return ()
        lax.fori_loop(0, GRP // 2 - 1, pair, ())

        j0 = GRP - 2
        j1 = GRP - 1
        gather_start(j1, rows1, gsem1)
        weights(j0)
        gather_wait(j0, rows0, gsem0)
        scale(j0, rows0)
        scatter_start(j0, rows0, ssem0)
        scatter_wait(j0, rows0, ssem0)
        weights(j1)
        gather_wait(j1, rows1, gsem1)
        scale(j1, rows1)
        scatter_start(j1, rows1, ssem1)
        scatter_wait(j1, rows1, ssem1)

        def drain(j, _):
            pltpu.make_async_copy(ex_v.at[j], d_sh.at[dsti.at[j]],
                                  dsem).wait()
            return ()
        lax.fori_loop(0, GRP, drain, ())
        return ()
    lax.fori_loop(0, CPW // GRP, group, ())

    plsc.subcore_barrier()
    pltpu.sync_copy(a_sh.at[pl.ds(base, ROWS_PER_TILE)],
                    ap_hbm.at[c, pl.ds(base, ROWS_PER_TILE)])
    pltpu.sync_copy(d_sh.at[pl.ds(base, ROWS_PER_TILE)],
                    dp_hbm.at[c, pl.ds(base, ROWS_PER_TILE)])


def _tc4_body(x_ref, ap_ref, dp_ref, out_ref):
    inv = 1.0 / jnp.maximum(dp_ref[0] + dp_ref[1], 1e-30)
    v = x_ref[...] + (ap_ref[0] + ap_ref[1]) * inv
    out_ref[...] = jnp.where(v > 0, v, jnp.exp(v) - 1.0)


def _tc_finish(x, ap, dp):
    grid = N // BLK1
    return pl.pallas_call(
        _tc4_body,
        grid=(grid,),
        in_specs=[
            pl.BlockSpec((BLK1, D), lambda i: (i, 0)),
            pl.BlockSpec((2, BLK1, D), lambda i: (0, i, 0)),
            pl.BlockSpec((2, BLK1, 1), lambda i: (0, i, 0)),
        ],
        out_specs=pl.BlockSpec((BLK1, D), lambda i: (i, 0)),
        out_shape=jax.ShapeDtypeStruct((N, D), jnp.float32),
    )(x, ap, dp)


_PAD_SRC = jnp.asarray(np.arange(EPAD - E) % N, dtype=jnp.int32)
_PAD_DST = jnp.asarray(N + np.arange(EPAD - E) % (NPAD - N), dtype=jnp.int32)


def kernel(x, edge_index, W_fc, W_l, W_r):
    src = edge_index[0]
    dst = edge_index[1]
    xp = jnp.zeros((NPAD, D), jnp.float32).at[:N].set(x)
    srcp = jnp.concatenate([src, _PAD_SRC]).reshape(NW * CPW, CHUNK)
    dstp = jnp.concatenate([dst, _PAD_DST]).reshape(NW * CPW, CHUNK)

    ftp, slp, srp = _tc_prepare(xp, W_fc, W_l, W_r)
    score, sp = _sc_scores(slp.reshape(-1), srp.reshape(-1), srcp, dstp)
    cp = _tc_offset(sp.reshape(2, NPAD // 128, 128))
    ap, dp = _sc_aggregate(cp.reshape(-1), srcp, dstp, score, ftp)
    return _tc_finish(x, ap, dp.reshape(2, NPAD, 1))


# speedup vs baseline: 1.0386x; 1.0386x over previous
import functools

import numpy as np

import jax
import jax.numpy as jnp
from jax import lax
from jax.experimental import pallas as pl
from jax.experimental.pallas import tpu as pltpu
from jax.experimental.pallas import tpu_sc as plsc

N = 10000
E = 320000
D = 128
NPAD = 10240
NW = 32
CPW = 80
CHUNK = 128
EPAD = NW * CPW * CHUNK
ROWS_PER_TILE = NPAD // 16
GRP = 16
BETA = 0.25
BLK = 1280
BLK1 = 1000

_mesh = plsc.VectorSubcoreMesh(core_axis_name="c", subcore_axis_name="s")


def _tc1_body(x_ref, wfc_ref, wl_ref, wr_ref, ft_ref, sl_ref, sr_ref):
    xb = x_ref[...]
    ft = lax.dot_general(xb, wfc_ref[...], (((1,), (1,)), ((), ())),
                         preferred_element_type=jnp.float32)
    ft_ref[...] = ft
    a1 = lax.dot_general(ft, wl_ref[...], (((1,), (1,)), ((), ())),
                         preferred_element_type=jnp.float32)
    a2 = lax.dot_general(ft, wr_ref[...], (((1,), (1,)), ((), ())),
                         preferred_element_type=jnp.float32)
    sl_ref[...] = jnp.sum(a1, axis=1).reshape(1, BLK // 128, 128)
    sr_ref[...] = jnp.sum(a2, axis=1).reshape(1, BLK // 128, 128)


def _tc_prepare(xp, w_fc, w_l, w_r):
    grid = NPAD // BLK
    return pl.pallas_call(
        _tc1_body,
        grid=(grid,),
        in_specs=[
            pl.BlockSpec((BLK, D), lambda i: (i, 0)),
            pl.BlockSpec((D, D), lambda i: (0, 0)),
            pl.BlockSpec((D, D), lambda i: (0, 0)),
            pl.BlockSpec((D, D), lambda i: (0, 0)),
        ],
        out_specs=[
            pl.BlockSpec((BLK, D), lambda i: (i, 0)),
            pl.BlockSpec((1, BLK // 128, 128), lambda i: (i, 0, 0)),
            pl.BlockSpec((1, BLK // 128, 128), lambda i: (i, 0, 0)),
        ],
        out_shape=[
            jax.ShapeDtypeStruct((NPAD, D), jnp.float32),
            jax.ShapeDtypeStruct((NPAD // BLK, BLK // 128, 128), jnp.float32),
            jax.ShapeDtypeStruct((NPAD // BLK, BLK // 128, 128), jnp.float32),
        ],
    )(xp, w_fc, w_l, w_r)


@functools.partial(
    pl.kernel,
    out_type=[
        jax.ShapeDtypeStruct((NW * CPW, CHUNK), jnp.float32),
        jax.ShapeDtypeStruct((2, NPAD), jnp.float32),
    ],
    mesh=_mesh,
    compiler_params=pltpu.CompilerParams(needs_layout_passes=False),
    scratch_types=[
        pltpu.VMEM((NPAD,), jnp.float32),
        pltpu.VMEM((NPAD,), jnp.float32),
        pltpu.VMEM((CPW, CHUNK), jnp.int32),
        pltpu.VMEM((CPW, CHUNK), jnp.int32),
        pltpu.VMEM((CPW, CHUNK), jnp.float32),
        pltpu.VMEM((CPW, CHUNK), jnp.float32),
        pltpu.VMEM((ROWS_PER_TILE,), jnp.float32),
        pltpu.VMEM_SHARED((NPAD,), jnp.float32),
        pltpu.SemaphoreType.DMA,
    ],
)
def _sc_scores(sl_hbm, sr_hbm, src_hbm, dst_hbm, score_hbm, sp_hbm,
               sl_v, sr_v, dsti, srci, score_v, expb_v, zbuf, s_sh, sem):
    c = lax.axis_index("c")
    s = lax.axis_index("s")
    wid = c * 16 + s
    pltpu.sync_copy(sl_hbm, sl_v)
    pltpu.sync_copy(sr_hbm, sr_v)
    pltpu.sync_copy(dst_hbm.at[pl.ds(wid * CPW, CPW)], dsti)
    pltpu.sync_copy(src_hbm.at[pl.ds(wid * CPW, CPW)], srci)

    def zero(i, _):
        zbuf[pl.ds(i * 16, 16)] = jnp.zeros((16,), jnp.float32)
        return ()
    lax.fori_loop(0, ROWS_PER_TILE // 16, zero, ())
    pltpu.sync_copy(zbuf, s_sh.at[pl.ds(s * ROWS_PER_TILE, ROWS_PER_TILE)])
    plsc.subcore_barrier()

    def chunk(j, _):
        for k in range(8):
            di = dsti[j, pl.ds(k * 16, 16)]
            si = srci[j, pl.ds(k * 16, 16)]
            sc = plsc.load_gather(sl_v, (di,)) + plsc.load_gather(sr_v, (si,))
            sc = jnp.maximum(sc, 0.01 * sc)
            score_v[j, pl.ds(k * 16, 16)] = sc
            expb_v[j, pl.ds(k * 16, 16)] = jnp.exp(BETA * sc)
        pltpu.async_copy(expb_v.at[j], s_sh.at[dsti.at[j]], sem, add=True)
        return ()
    lax.fori_loop(0, CPW, chunk, ())

    def drain(j, _):
        pltpu.make_async_copy(expb_v.at[j], s_sh.at[dsti.at[j]], sem).wait()
        return ()
    lax.fori_loop(0, CPW, drain, ())

    pltpu.sync_copy(score_v, score_hbm.at[pl.ds(wid * CPW, CPW)])
    plsc.subcore_barrier()
    pltpu.sync_copy(s_sh.at[pl.ds(s * ROWS_PER_TILE, ROWS_PER_TILE)],
                    sp_hbm.at[c, pl.ds(s * ROWS_PER_TILE, ROWS_PER_TILE)])


def _tc2_body(sp_ref, c_ref):
    c_ref[...] = jnp.log(sp_ref[0] + sp_ref[1]) * (1.0 / BETA)


def _tc_offset(sp):
    return pl.pallas_call(
        _tc2_body,
        out_shape=jax.ShapeDtypeStruct((NPAD // 128, 128), jnp.float32),
    )(sp)


@functools.partial(
    pl.kernel,
    out_type=[
        jax.ShapeDtypeStruct((NW * CPW, CHUNK), jnp.float32),
        jax.ShapeDtypeStruct((2, NPAD), jnp.float32),
    ],
    mesh=_mesh,
    compiler_params=pltpu.CompilerParams(needs_layout_passes=False),
    scratch_types=[
        pltpu.VMEM((NPAD,), jnp.float32),
        pltpu.VMEM((CPW, CHUNK), jnp.int32),
        pltpu.VMEM((CPW, CHUNK), jnp.float32),
        pltpu.VMEM((CPW, CHUNK), jnp.float32),
        pltpu.VMEM((ROWS_PER_TILE,), jnp.float32),
        pltpu.VMEM_SHARED((NPAD,), jnp.float32),
        pltpu.SemaphoreType.DMA,
    ],
)
def _sc_denom(c_hbm, dst_hbm, score_hbm, ex_hbm, dp_hbm,
              c_v, dsti, score_v, ex_v, zbuf, d_sh, sem):
    c = lax.axis_index("c")
    s = lax.axis_index("s")
    wid = c * 16 + s
    pltpu.sync_copy(c_hbm, c_v)
    pltpu.sync_copy(dst_hbm.at[pl.ds(wid * CPW, CPW)], dsti)
    pltpu.sync_copy(score_hbm.at[pl.ds(wid * CPW, CPW)], score_v)

    def zero(i, _):
        zbuf[pl.ds(i * 16, 16)] = jnp.zeros((16,), jnp.float32)
        return ()
    lax.fori_loop(0, ROWS_PER_TILE // 16, zero, ())
    pltpu.sync_copy(zbuf, d_sh.at[pl.ds(s * ROWS_PER_TILE, ROWS_PER_TILE)])
    plsc.subcore_barrier()

    def chunk(j, _):
        for k in range(8):
            di = dsti[j, pl.ds(k * 16, 16)]
            cg = plsc.load_gather(c_v, (di,))
            sc = score_v[j, pl.ds(k * 16, 16)]
            ex_v[j, pl.ds(k * 16, 16)] = jnp.exp(sc - cg)
        pltpu.async_copy(ex_v.at[j], d_sh.at[dsti.at[j]], sem, add=True)
        return ()
    lax.fori_loop(0, CPW, chunk, ())

    def drain(j, _):
        pltpu.make_async_copy(ex_v.at[j], d_sh.at[dsti.at[j]], sem).wait()
        return ()
    lax.fori_loop(0, CPW, drain, ())

    pltpu.sync_copy(ex_v, ex_hbm.at[pl.ds(wid * CPW, CPW)])
    plsc.subcore_barrier()
    pltpu.sync_copy(d_sh.at[pl.ds(s * ROWS_PER_TILE, ROWS_PER_TILE)],
                    dp_hbm.at[c, pl.ds(s * ROWS_PER_TILE, ROWS_PER_TILE)])


@functools.partial(
    pl.kernel,
    out_type=jax.ShapeDtypeStruct((2, NPAD, D), jnp.float32),
    mesh=_mesh,
    compiler_params=pltpu.CompilerParams(needs_layout_passes=False),
    scratch_types=[
        pltpu.VMEM((GRP, CHUNK), jnp.int32),
        pltpu.VMEM((GRP, CHUNK), jnp.int32),
        pltpu.VMEM((GRP, CHUNK), jnp.float32),
        pltpu.VMEM((CHUNK, D), jnp.float32),
        pltpu.VMEM((CHUNK, D), jnp.float32),
        pltpu.VMEM_SHARED((NPAD, D), jnp.float32),
        pltpu.SemaphoreType.DMA,
        pltpu.SemaphoreType.DMA,
        pltpu.SemaphoreType.DMA,
        pltpu.SemaphoreType.DMA,
    ],
)
def _sc_aggregate(src_hbm, dst_hbm, ex_hbm, ft_hbm, ap_hbm,
                  dsti, srci, ex_v, rows0, rows1, a_sh,
                  gsem0, gsem1, ssem0, ssem1):
    c = lax.axis_index("c")
    s = lax.axis_index("s")
    wid = c * 16 + s

    def zero_row(i, _):
        for q in range(D // 16):
            rows0[i, pl.ds(q * 16, 16)] = jnp.zeros((16,), jnp.float32)
        return ()
    lax.fori_loop(0, CHUNK, zero_row, ())
    base = s * ROWS_PER_TILE
    for i in range(ROWS_PER_TILE // CHUNK):
        pltpu.sync_copy(rows0, a_sh.at[pl.ds(base + i * CHUNK, CHUNK)])
    plsc.subcore_barrier()

    def gather_start(j, buf, sem):
        pltpu.async_copy(ft_hbm.at[srci.at[j]], buf, sem)

    def gather_wait(j, buf, sem):
        pltpu.make_async_copy(ft_hbm.at[srci.at[j]], buf, sem).wait()

    def scale(j, buf):
        def scale_grp(k, _):
            e16 = ex_v[j, pl.ds(k * 16, 16)]
            for lane in range(16):
                es = e16[lane]
                r = k * 16 + lane
                for q in range(D // 16):
                    buf[r, pl.ds(q * 16, 16)] = buf[r, pl.ds(q * 16, 16)] * es
            return ()
        lax.fori_loop(0, CHUNK // 16, scale_grp, ())

    def scatter_start(j, buf, sem):
        pltpu.async_copy(buf, a_sh.at[dsti.at[j]], sem, add=True)

    def scatter_wait(j, buf, sem):
        pltpu.make_async_copy(buf, a_sh.at[dsti.at[j]], sem).wait()

    def group(g, _):
        gb = wid * CPW + g * GRP
        pltpu.sync_copy(dst_hbm.at[pl.ds(gb, GRP)], dsti)
        pltpu.sync_copy(src_hbm.at[pl.ds(gb, GRP)], srci)
        pltpu.sync_copy(ex_hbm.at[pl.ds(gb, GRP)], ex_v)
        gather_start(0, rows0, gsem0)

        def pair(i, _):
            j0 = 2 * i
            j1 = j0 + 1
            gather_start(j1, rows1, gsem1)
            gather_wait(j0, rows0, gsem0)
            scale(j0, rows0)
            scatter_start(j0, rows0, ssem0)
            scatter_wait(j0, rows0, ssem0)
            gather_start(j0 + 2, rows0, gsem0)
            gather_wait(j1, rows1, gsem1)
            scale(j1, rows1)
            scatter_start(j1, rows1, ssem1)
            scatter_wait(j1, rows1, ssem1)
            return ()
        lax.fori_loop(0, GRP // 2 - 1, pair, ())

        j0 = GRP - 2
        j1 = GRP - 1
        gather_start(j1, rows1, gsem1)
        gather_wait(j0, rows0, gsem0)
        scale(j0, rows0)
        scatter_start(j0, rows0, ssem0)
        scatter_wait(j0, rows0, ssem0)
        gather_wait(j1, rows1, gsem1)
        scale(j1, rows1)
        scatter_start(j1, rows1, ssem1)
        scatter_wait(j1, rows1, ssem1)
        return ()
    lax.fori_loop(0, CPW // GRP, group, ())

    plsc.subcore_barrier()
    pltpu.sync_copy(a_sh.at[pl.ds(base, ROWS_PER_TILE)],
                    ap_hbm.at[c, pl.ds(base, ROWS_PER_TILE)])


def _tc4_body(x_ref, ap_ref, dp_ref, out_ref):
    inv = 1.0 / jnp.maximum(dp_ref[0] + dp_ref[1], 1e-30)
    v = x_ref[...] + (ap_ref[0] + ap_ref[1]) * inv
    out_ref[...] = jnp.where(v > 0, v, jnp.exp(v) - 1.0)


def _tc_finish(x, ap, dp):
    grid = N // BLK1
    return pl.pallas_call(
        _tc4_body,
        grid=(grid,),
        in_specs=[
            pl.BlockSpec((BLK1, D), lambda i: (i, 0)),
            pl.BlockSpec((2, BLK1, D), lambda i: (0, i, 0)),
            pl.BlockSpec((2, BLK1, 1), lambda i: (0, i, 0)),
        ],
        out_specs=pl.BlockSpec((BLK1, D), lambda i: (i, 0)),
        out_shape=jax.ShapeDtypeStruct((N, D), jnp.float32),
    )(x, ap, dp)


_PAD_SRC = jnp.asarray(np.arange(EPAD - E) % N, dtype=jnp.int32)
_PAD_DST = jnp.asarray(N + np.arange(EPAD - E) % (NPAD - N), dtype=jnp.int32)


def kernel(x, edge_index, W_fc, W_l, W_r):
    src = edge_index[0]
    dst = edge_index[1]
    xp = jnp.zeros((NPAD, D), jnp.float32).at[:N].set(x)
    srcp = jnp.concatenate([src, _PAD_SRC]).reshape(NW * CPW, CHUNK)
    dstp = jnp.concatenate([dst, _PAD_DST]).reshape(NW * CPW, CHUNK)

    ftp, slp, srp = _tc_prepare(xp, W_fc, W_l, W_r)
    score, sp = _sc_scores(slp.reshape(-1), srp.reshape(-1), srcp, dstp)
    cp = _tc_offset(sp.reshape(2, NPAD // 128, 128))
    ex, dp = _sc_denom(cp.reshape(-1), dstp, score)
    ap = _sc_aggregate(srcp, dstp, ex, ftp)
    return _tc_finish(x, ap, dp.reshape(2, NPAD, 1))


# speedup vs baseline: 1.0877x; 1.0473x over previous
import functools

import numpy as np

import jax
import jax.numpy as jnp
from jax import lax
from jax.experimental import pallas as pl
from jax.experimental.pallas import tpu as pltpu
from jax.experimental.pallas import tpu_sc as plsc

N = 10000
E = 320000
D = 128
NPAD = 10240
NW = 32
CPW = 80
CHUNK = 128
EPAD = NW * CPW * CHUNK
ROWS_PER_TILE = NPAD // 16
GRP = 40
BETA = 0.25
BLK = 1280
BLK1 = 1000

_mesh = plsc.VectorSubcoreMesh(core_axis_name="c", subcore_axis_name="s")


def _tc1_body(x_ref, wfc_ref, wl_ref, wr_ref, ft_ref, sl_ref, sr_ref):
    xb = x_ref[...]
    ft = lax.dot_general(xb, wfc_ref[...], (((1,), (1,)), ((), ())),
                         preferred_element_type=jnp.float32)
    ft_ref[...] = ft
    a1 = lax.dot_general(ft, wl_ref[...], (((1,), (1,)), ((), ())),
                         preferred_element_type=jnp.float32)
    a2 = lax.dot_general(ft, wr_ref[...], (((1,), (1,)), ((), ())),
                         preferred_element_type=jnp.float32)
    sl_ref[...] = jnp.sum(a1, axis=1).reshape(1, BLK // 128, 128)
    sr_ref[...] = jnp.sum(a2, axis=1).reshape(1, BLK // 128, 128)


def _tc_prepare(xp, w_fc, w_l, w_r):
    grid = NPAD // BLK
    return pl.pallas_call(
        _tc1_body,
        grid=(grid,),
        in_specs=[
            pl.BlockSpec((BLK, D), lambda i: (i, 0)),
            pl.BlockSpec((D, D), lambda i: (0, 0)),
            pl.BlockSpec((D, D), lambda i: (0, 0)),
            pl.BlockSpec((D, D), lambda i: (0, 0)),
        ],
        out_specs=[
            pl.BlockSpec((BLK, D), lambda i: (i, 0)),
            pl.BlockSpec((1, BLK // 128, 128), lambda i: (i, 0, 0)),
            pl.BlockSpec((1, BLK // 128, 128), lambda i: (i, 0, 0)),
        ],
        out_shape=[
            jax.ShapeDtypeStruct((NPAD, D), jnp.float32),
            jax.ShapeDtypeStruct((NPAD // BLK, BLK // 128, 128), jnp.float32),
            jax.ShapeDtypeStruct((NPAD // BLK, BLK // 128, 128), jnp.float32),
        ],
    )(xp, w_fc, w_l, w_r)


@functools.partial(
    pl.kernel,
    out_type=[
        jax.ShapeDtypeStruct((NW * CPW, CHUNK), jnp.float32),
        jax.ShapeDtypeStruct((2, NPAD), jnp.float32),
    ],
    mesh=_mesh,
    compiler_params=pltpu.CompilerParams(needs_layout_passes=False),
    scratch_types=[
        pltpu.VMEM((NPAD,), jnp.float32),
        pltpu.VMEM((NPAD,), jnp.float32),
        pltpu.VMEM((CPW, CHUNK), jnp.int32),
        pltpu.VMEM((CPW, CHUNK), jnp.int32),
        pltpu.VMEM((CPW, CHUNK), jnp.float32),
        pltpu.VMEM((CPW, CHUNK), jnp.float32),
        pltpu.VMEM((ROWS_PER_TILE,), jnp.float32),
        pltpu.VMEM_SHARED((NPAD,), jnp.float32),
        pltpu.SemaphoreType.DMA,
    ],
)
def _sc_scores(sl_hbm, sr_hbm, src_hbm, dst_hbm, score_hbm, sp_hbm,
               sl_v, sr_v, dsti, srci, score_v, expb_v, zbuf, s_sh, sem):
    c = lax.axis_index("c")
    s = lax.axis_index("s")
    wid = c * 16 + s
    pltpu.sync_copy(sl_hbm, sl_v)
    pltpu.sync_copy(sr_hbm, sr_v)
    pltpu.sync_copy(dst_hbm.at[pl.ds(wid * CPW, CPW)], dsti)
    pltpu.sync_copy(src_hbm.at[pl.ds(wid * CPW, CPW)], srci)

    def zero(i, _):
        zbuf[pl.ds(i * 16, 16)] = jnp.zeros((16,), jnp.float32)
        return ()
    lax.fori_loop(0, ROWS_PER_TILE // 16, zero, ())
    pltpu.sync_copy(zbuf, s_sh.at[pl.ds(s * ROWS_PER_TILE, ROWS_PER_TILE)])
    plsc.subcore_barrier()

    def chunk(j, _):
        for k in range(8):
            di = dsti[j, pl.ds(k * 16, 16)]
            si = srci[j, pl.ds(k * 16, 16)]
            sc = plsc.load_gather(sl_v, (di,)) + plsc.load_gather(sr_v, (si,))
            sc = jnp.maximum(sc, 0.01 * sc)
            score_v[j, pl.ds(k * 16, 16)] = sc
            expb_v[j, pl.ds(k * 16, 16)] = jnp.exp(BETA * sc)
        pltpu.async_copy(expb_v.at[j], s_sh.at[dsti.at[j]], sem, add=True)
        return ()
    lax.fori_loop(0, CPW, chunk, ())

    def drain(j, _):
        pltpu.make_async_copy(expb_v.at[j], s_sh.at[dsti.at[j]], sem).wait()
        return ()
    lax.fori_loop(0, CPW, drain, ())

    pltpu.sync_copy(score_v, score_hbm.at[pl.ds(wid * CPW, CPW)])
    plsc.subcore_barrier()
    pltpu.sync_copy(s_sh.at[pl.ds(s * ROWS_PER_TILE, ROWS_PER_TILE)],
                    sp_hbm.at[c, pl.ds(s * ROWS_PER_TILE, ROWS_PER_TILE)])


def _tc2_body(sp_ref, c_ref):
    c_ref[...] = jnp.log(sp_ref[0] + sp_ref[1]) * (1.0 / BETA)


def _tc_offset(sp):
    return pl.pallas_call(
        _tc2_body,
        out_shape=jax.ShapeDtypeStruct((NPAD // 128, 128), jnp.float32),
    )(sp)


@functools.partial(
    pl.kernel,
    out_type=[
        jax.ShapeDtypeStruct((NW * CPW, CHUNK), jnp.float32),
        jax.ShapeDtypeStruct((2, NPAD), jnp.float32),
    ],
    mesh=_mesh,
    compiler_params=pltpu.CompilerParams(needs_layout_passes=False),
    scratch_types=[
        pltpu.VMEM((NPAD,), jnp.float32),
        pltpu.VMEM((CPW, CHUNK), jnp.int32),
        pltpu.VMEM((CPW, CHUNK), jnp.float32),
        pltpu.VMEM((CPW, CHUNK), jnp.float32),
        pltpu.VMEM((ROWS_PER_TILE,), jnp.float32),
        pltpu.VMEM_SHARED((NPAD,), jnp.float32),
        pltpu.SemaphoreType.DMA,
    ],
)
def _sc_denom(c_hbm, dst_hbm, score_hbm, ex_hbm, dp_hbm,
              c_v, dsti, score_v, ex_v, zbuf, d_sh, sem):
    c = lax.axis_index("c")
    s = lax.axis_index("s")
    wid = c * 16 + s
    pltpu.sync_copy(c_hbm, c_v)
    pltpu.sync_copy(dst_hbm.at[pl.ds(wid * CPW, CPW)], dsti)
    pltpu.sync_copy(score_hbm.at[pl.ds(wid * CPW, CPW)], score_v)

    def zero(i, _):
        zbuf[pl.ds(i * 16, 16)] = jnp.zeros((16,), jnp.float32)
        return ()
    lax.fori_loop(0, ROWS_PER_TILE // 16, zero, ())
    pltpu.sync_copy(zbuf, d_sh.at[pl.ds(s * ROWS_PER_TILE, ROWS_PER_TILE)])
    plsc.subcore_barrier()

    def chunk(j, _):
        for k in range(8):
            di = dsti[j, pl.ds(k * 16, 16)]
            cg = plsc.load_gather(c_v, (di,))
            sc = score_v[j, pl.ds(k * 16, 16)]
            ex_v[j, pl.ds(k * 16, 16)] = jnp.exp(sc - cg)
        pltpu.async_copy(ex_v.at[j], d_sh.at[dsti.at[j]], sem, add=True)
        return ()
    lax.fori_loop(0, CPW, chunk, ())

    def drain(j, _):
        pltpu.make_async_copy(ex_v.at[j], d_sh.at[dsti.at[j]], sem).wait()
        return ()
    lax.fori_loop(0, CPW, drain, ())

    pltpu.sync_copy(ex_v, ex_hbm.at[pl.ds(wid * CPW, CPW)])
    plsc.subcore_barrier()
    pltpu.sync_copy(d_sh.at[pl.ds(s * ROWS_PER_TILE, ROWS_PER_TILE)],
                    dp_hbm.at[c, pl.ds(s * ROWS_PER_TILE, ROWS_PER_TILE)])


@functools.partial(
    pl.kernel,
    out_type=jax.ShapeDtypeStruct((2, NPAD, D), jnp.float32),
    mesh=_mesh,
    compiler_params=pltpu.CompilerParams(needs_layout_passes=False),
    scratch_types=[
        pltpu.VMEM((GRP, CHUNK), jnp.int32),
        pltpu.VMEM((GRP, CHUNK), jnp.int32),
        pltpu.VMEM((GRP, CHUNK), jnp.float32),
        pltpu.VMEM((CHUNK, D), jnp.float32),
        pltpu.VMEM((CHUNK, D), jnp.float32),
        pltpu.VMEM_SHARED((NPAD, D), jnp.float32),
        pltpu.SemaphoreType.DMA,
        pltpu.SemaphoreType.DMA,
        pltpu.SemaphoreType.DMA,
        pltpu.SemaphoreType.DMA,
    ],
)
def _sc_aggregate(src_hbm, dst_hbm, ex_hbm, ft_hbm, ap_hbm,
                  dsti, srci, ex_v, rows0, rows1, a_sh,
                  gsem0, gsem1, ssem0, ssem1):
    c = lax.axis_index("c")
    s = lax.axis_index("s")
    wid = c * 16 + s

    def zero_row(i, _):
        for q in range(D // 16):
            rows0[i, pl.ds(q * 16, 16)] = jnp.zeros((16,), jnp.float32)
        return ()
    lax.fori_loop(0, CHUNK, zero_row, ())
    base = s * ROWS_PER_TILE
    for i in range(ROWS_PER_TILE // CHUNK):
        pltpu.sync_copy(rows0, a_sh.at[pl.ds(base + i * CHUNK, CHUNK)])
    plsc.subcore_barrier()

    def gather_start(j, buf, sem):
        pltpu.async_copy(ft_hbm.at[srci.at[j]], buf, sem)

    def gather_wait(j, buf, sem):
        pltpu.make_async_copy(ft_hbm.at[srci.at[j]], buf, sem).wait()

    def scale(j, buf):
        def scale_grp(k, _):
            e16 = ex_v[j, pl.ds(k * 16, 16)]
            for lane in range(16):
                es = e16[lane]
                r = k * 16 + lane
                for q in range(D // 16):
                    buf[r, pl.ds(q * 16, 16)] = buf[r, pl.ds(q * 16, 16)] * es
            return ()
        lax.fori_loop(0, CHUNK // 16, scale_grp, ())

    def scatter_start(j, buf, sem):
        pltpu.async_copy(buf, a_sh.at[dsti.at[j]], sem, add=True)

    def scatter_wait(j, buf, sem):
        pltpu.make_async_copy(buf, a_sh.at[dsti.at[j]], sem).wait()

    def group(g, _):
        gb = wid * CPW + g * GRP
        pltpu.sync_copy(dst_hbm.at[pl.ds(gb, GRP)], dsti)
        pltpu.sync_copy(src_hbm.at[pl.ds(gb, GRP)], srci)
        pltpu.sync_copy(ex_hbm.at[pl.ds(gb, GRP)], ex_v)
        gather_start(0, rows0, gsem0)

        def pair(i, _):
            j0 = 2 * i
            j1 = j0 + 1
            gather_start(j1, rows1, gsem1)
            gather_wait(j0, rows0, gsem0)
            scale(j0, rows0)
            scatter_start(j0, rows0, ssem0)
            scatter_wait(j0, rows0, ssem0)
            gather_start(j0 + 2, rows0, gsem0)
            gather_wait(j1, rows1, gsem1)
            scale(j1, rows1)
            scatter_start(j1, rows1, ssem1)
            scatter_wait(j1, rows1, ssem1)
            return ()
        lax.fori_loop(0, GRP // 2 - 1, pair, ())

        j0 = GRP - 2
        j1 = GRP - 1
        gather_start(j1, rows1, gsem1)
        gather_wait(j0, rows0, gsem0)
        scale(j0, rows0)
        scatter_start(j0, rows0, ssem0)
        scatter_wait(j0, rows0, ssem0)
        gather_wait(j1, rows1, gsem1)
        scale(j1, rows1)
        scatter_start(j1, rows1, ssem1)
        scatter_wait(j1, rows1, ssem1)
        return ()
    lax.fori_loop(0, CPW // GRP, group, ())

    plsc.subcore_barrier()
    pltpu.sync_copy(a_sh.at[pl.ds(base, ROWS_PER_TILE)],
                    ap_hbm.at[c, pl.ds(base, ROWS_PER_TILE)])


def _tc4_body(x_ref, ap_ref, dp_ref, out_ref):
    inv = 1.0 / jnp.maximum(dp_ref[0] + dp_ref[1], 1e-30)
    v = x_ref[...] + (ap_ref[0] + ap_ref[1]) * inv
    out_ref[...] = jnp.where(v > 0, v, jnp.exp(v) - 1.0)


def _tc_finish(x, ap, dp):
    grid = N // BLK1
    return pl.pallas_call(
        _tc4_body,
        grid=(grid,),
        in_specs=[
            pl.BlockSpec((BLK1, D), lambda i: (i, 0)),
            pl.BlockSpec((2, BLK1, D), lambda i: (0, i, 0)),
            pl.BlockSpec((2, BLK1, 1), lambda i: (0, i, 0)),
        ],
        out_specs=pl.BlockSpec((BLK1, D), lambda i: (i, 0)),
        out_shape=jax.ShapeDtypeStruct((N, D), jnp.float32),
    )(x, ap, dp)


_PAD_SRC = jnp.asarray(np.arange(EPAD - E) % N, dtype=jnp.int32)
_PAD_DST = jnp.asarray(N + np.arange(EPAD - E) % (NPAD - N), dtype=jnp.int32)


def kernel(x, edge_index, W_fc, W_l, W_r):
    src = edge_index[0]
    dst = edge_index[1]
    xp = jnp.zeros((NPAD, D), jnp.float32).at[:N].set(x)
    srcp = jnp.concatenate([src, _PAD_SRC]).reshape(NW * CPW, CHUNK)
    dstp = jnp.concatenate([dst, _PAD_DST]).reshape(NW * CPW, CHUNK)

    ftp, slp, srp = _tc_prepare(xp, W_fc, W_l, W_r)
    score, sp = _sc_scores(slp.reshape(-1), srp.reshape(-1), srcp, dstp)
    cp = _tc_offset(sp.reshape(2, NPAD // 128, 128))
    ex, dp = _sc_denom(cp.reshape(-1), dstp, score)
    ap = _sc_aggregate(srcp, dstp, ex, ftp)
    return _tc_finish(x, ap, dp.reshape(2, NPAD, 1))
